# baseline (speedup 1.0000x reference)
import functools

import jax
import jax.numpy as jnp
from jax import lax
from jax.experimental import pallas as pl
from jax.experimental.pallas import tpu as pltpu
from jax.experimental.pallas import tpu_sc as plsc

_NENTS = 1000000
_DIM = 64
_B = 16384
_TB = 2 * _B
_MARGIN = 1.0
_L2REG = 0.1

_NW = 32
_TPW = _TB // _NW
_G = 16
_NG = _TPW // _G
_DEPTH = 4


def _sc_body(hidx_hbm, ridx_hbm, tidx_hbm, ents_hbm, rtab_hbm, out_hbm,
             hidx, ridx, tidx, rtab, hbuf, tbuf, outbuf, *sems):
    sem_h = sems[:_DEPTH]
    sem_t = sems[_DEPTH:]
    c = lax.axis_index("c")
    s = lax.axis_index("s")
    wid = s * 2 + c
    base = wid * _TPW
    pltpu.sync_copy(hidx_hbm.at[pl.ds(base, _TPW)], hidx)
    pltpu.sync_copy(ridx_hbm.at[pl.ds(base, _TPW)], ridx)
    pltpu.sync_copy(tidx_hbm.at[pl.ds(base, _TPW)], tidx)
    pltpu.sync_copy(rtab_hbm, rtab)

    def _fire(g, u):
        ivh = hidx[pl.ds(g * _G, _G)]
        ivt = tidx[pl.ds(g * _G, _G)]
        for l in range(_G):
            pltpu.async_copy(
                ents_hbm.at[pl.ds(ivh[l], 1)], hbuf.at[u, pl.ds(l, 1)],
                sem_h[u])
            pltpu.async_copy(
                ents_hbm.at[pl.ds(ivt[l], 1)], tbuf.at[u, pl.ds(l, 1)],
                sem_t[u])

    for u in range(_DEPTH):
        _fire(u, u)

    def _quad(jj, carry):
        for u in range(_DEPTH):
            g = _DEPTH * jj + u
            pltpu.make_async_copy(
                ents_hbm.at[pl.ds(0, _G)], hbuf.at[u], sem_h[u]).wait()
            pltpu.make_async_copy(
                ents_hbm.at[pl.ds(0, _G)], tbuf.at[u], sem_t[u]).wait()

            @pl.when(g + _DEPTH < _NG)
            def _():
                _fire(g + _DEPTH, u)

            ivr = ridx[pl.ds(g * _G, _G)]
            for l in range(_G):
                ri = ivr[l]
                row = lax.shift_right_logical(ri, 1)
                col0 = lax.mul(lax.rem(ri, 2), _DIM)
                acc = jnp.zeros((16,), jnp.float32)
                for k in range(_DIM // 16):
                    hv = hbuf[u, l, pl.ds(k * 16, 16)]
                    tv = tbuf[u, l, pl.ds(k * 16, 16)]
                    rv = rtab[row, pl.ds(col0 + k * 16, 16)]
                    d = (hv + rv) - tv
                    acc = acc + d * d
                outbuf[2 * g + (l // 8), pl.ds((l % 8) * 16, 16)] = acc
        return carry

    lax.fori_loop(0, _NG // _DEPTH, _quad, 0)
    pltpu.sync_copy(outbuf, out_hbm.at[pl.ds(wid * 128, 128)])


@functools.cache
def _sc_scores():
    return functools.partial(
        pl.kernel,
        mesh=plsc.VectorSubcoreMesh(core_axis_name="c", subcore_axis_name="s"),
        compiler_params=pltpu.CompilerParams(needs_layout_passes=False),
        cost_estimate=pl.CostEstimate(
            flops=8_000_000, bytes_accessed=20_000_000, transcendentals=0),
        out_type=jax.ShapeDtypeStruct((_TB // 8, 128), jnp.float32),
        scratch_types=[
            pltpu.VMEM((_TPW,), jnp.int32),
            pltpu.VMEM((_TPW,), jnp.int32),
            pltpu.VMEM((_TPW,), jnp.int32),
            pltpu.VMEM((500, 128), jnp.float32),
            pltpu.VMEM((_DEPTH, _G, _DIM), jnp.float32),
            pltpu.VMEM((_DEPTH, _G, _DIM), jnp.float32),
            pltpu.VMEM((128, 128), jnp.float32),
        ] + [pltpu.SemaphoreType.DMA] * (2 * _DEPTH),
    )(_sc_body)


_S3N = 4
_S3STEPS = 25
_B3 = _NENTS // 8 // _S3N // _S3STEPS


def _sweep_body(*refs):
    out_ref = refs[-1]

    @pl.when(pl.program_id(0) == 0)
    def _():
        out_ref[0, 0] = 0.0

    ones = jnp.ones((1, _DIM), jnp.float32)
    tot = jnp.float32(0.0)
    for ref in refs[:-1]:
        x = ref[...].reshape(_B3 * 8, _DIM)
        y = x * x
        s2 = lax.dot_general(ones, y, (((1,), (1,)), ((), ())),
                             preferred_element_type=jnp.float32)
        r = jnp.sqrt(jnp.maximum(s2, 1.0)) - 1.0
        tot = tot + jnp.sum(r)
    out_ref[0, 0] += tot


_sweep_call = pl.pallas_call(
    _sweep_body,
    grid=(_S3STEPS,),
    in_specs=[
        pl.BlockSpec((_B3, 8, _DIM), lambda i, k=k: (k * _S3STEPS + i, 0, 0))
        for k in range(_S3N)
    ],
    out_specs=pl.BlockSpec(memory_space=pltpu.SMEM),
    out_shape=jax.ShapeDtypeStruct((1, 1), jnp.float32),
)


def _sweep(ents_w):
    e3v = ents_w.reshape(_NENTS // 8, 8, _DIM)
    return _sweep_call(*([e3v] * _S3N))


_PR = _TB // 8


def _final_body(part_ref, reg_ref, out_ref):
    x = part_ref[...]
    rows = lax.broadcasted_iota(jnp.int32, (128, 8), 0)
    cols = lax.broadcasted_iota(jnp.int32, (128, 8), 1)
    m = (rows // 16 == cols).astype(jnp.float32)
    sc2 = jnp.dot(x, m, preferred_element_type=jnp.float32)
    scores = jnp.sqrt(sc2)
    gold = scores[: _PR // 2]
    corrupt = scores[_PR // 2:]
    rank = jnp.sum(jnp.maximum(_MARGIN + gold - corrupt, 0.0))
    out_ref[0, 0] = rank + _L2REG * reg_ref[0, 0] + _L2REG * jnp.sum(gold)


_final = pl.pallas_call(
    _final_body,
    in_specs=[
        pl.BlockSpec((_PR, 128), lambda: (0, 0)),
        pl.BlockSpec(memory_space=pltpu.SMEM),
    ],
    out_specs=pl.BlockSpec(memory_space=pltpu.SMEM),
    out_shape=jax.ShapeDtypeStruct((1, 1), jnp.float32),
)


def kernel(heads, rels, tails, sources, heads_bad, rels_bad, tails_bad,
           sources_bad, ents_w, rels_w):
    del sources, sources_bad
    hidx = jnp.concatenate([heads, heads_bad]).astype(jnp.int32)
    ridx = jnp.concatenate([rels, rels_bad]).astype(jnp.int32)
    tidx = jnp.concatenate([tails, tails_bad]).astype(jnp.int32)
    rtab = rels_w.reshape(500, 128)
    part = _sc_scores()(hidx, ridx, tidx, ents_w, rtab)
    reg = _sweep(ents_w)
    out = _final(part, reg)
    return out[0, 0]
